# baseline (speedup 1.0000x reference)
import jax
import jax.numpy as jnp
from jax.experimental import pallas as pl
from jax.experimental.pallas import tpu as pltpu

_C = 64
_OUT = 64


_B = 2
_MT = 32


def _body(x_ref, wr_ref, wi_ref, o_ref):
    w_eff = 2.0 * (wr_ref[0:64, :] - wi_ref[64:128, :] - wr_ref[128:192, :]
                   + wi_ref[192:256, :] + wr_ref[256:320, :])
    for b in range(o_ref.shape[0]):
        xb = x_ref[b]
        for i in range(xb.shape[1]):
            o_ref[b, i] = jax.lax.dot_general(
                xb[:, i, :], w_eff, (((0,), (0,)), ((), ())),
                preferred_element_type=jnp.float32)


def kernel(x, real_weights, imag_weights):
    N, C, m, _ = x.shape
    B = _B if N % _B == 0 else 1
    mt = _MT if m % _MT == 0 else m
    out = pl.pallas_call(
        _body,
        grid=(N // B, m // mt),
        in_specs=[
            pl.BlockSpec((B, C, mt, m), lambda n, j: (n, 0, j, 0)),
            pl.BlockSpec(real_weights.shape, lambda n, j: (0, 0)),
            pl.BlockSpec(imag_weights.shape, lambda n, j: (0, 0)),
        ],
        out_specs=pl.BlockSpec((B, mt, m, _OUT), lambda n, j: (n, j, 0, 0)),
        out_shape=jax.ShapeDtypeStruct((N, m, m, _OUT), jnp.float32),
        compiler_params=pltpu.CompilerParams(
            dimension_semantics=("parallel", "parallel")),
    )(x, real_weights, imag_weights)
    return out


# speedup vs baseline: 1.0925x; 1.0925x over previous
import jax
import jax.numpy as jnp
from jax.experimental import pallas as pl
from jax.experimental.pallas import tpu as pltpu

_C = 64
_OUT = 64


_B = 2
_MT = 64


def _body(x_ref, wr_ref, wi_ref, o_ref):
    w_eff = 2.0 * (wr_ref[0:64, :] - wi_ref[64:128, :] - wr_ref[128:192, :]
                   + wi_ref[192:256, :] + wr_ref[256:320, :])
    for b in range(o_ref.shape[0]):
        xb = x_ref[b]
        for i in range(xb.shape[1]):
            o_ref[b, i] = jax.lax.dot_general(
                xb[:, i, :], w_eff, (((0,), (0,)), ((), ())),
                preferred_element_type=jnp.float32)


def kernel(x, real_weights, imag_weights):
    N, C, m, _ = x.shape
    B = _B if N % _B == 0 else 1
    mt = _MT if m % _MT == 0 else m
    out = pl.pallas_call(
        _body,
        grid=(N // B, m // mt),
        in_specs=[
            pl.BlockSpec((B, C, mt, m), lambda n, j: (n, 0, j, 0)),
            pl.BlockSpec(real_weights.shape, lambda n, j: (0, 0)),
            pl.BlockSpec(imag_weights.shape, lambda n, j: (0, 0)),
        ],
        out_specs=pl.BlockSpec((B, mt, m, _OUT), lambda n, j: (n, j, 0, 0)),
        out_shape=jax.ShapeDtypeStruct((N, m, m, _OUT), jnp.float32),
        compiler_params=pltpu.CompilerParams(
            dimension_semantics=("parallel", "parallel")),
    )(x, real_weights, imag_weights)
    return out
